# Initial kernel scaffold; baseline (speedup 1.0000x reference)
import jax
import jax.numpy as jnp
from jax.experimental import pallas as pl


def kernel(features, adj, weight, weight2, w_m1, b_m1, w_m2, b_m2, w1, b1, w2, b2):
    raise NotImplementedError("write your pallas kernel here")


# speedup vs baseline: 1.0003x; 1.0003x over previous
import functools

import jax
import jax.numpy as jnp
from jax.experimental import pallas as pl
from jax.experimental.pallas import tpu as pltpu


def _pass1_body(adj_ref, feat_ref, w_ref, w2_ref, w1b_ref,
                c1w2_ref, p1_ref, fw_s):
    i = pl.program_id(0)

    @pl.when(i == 0)
    def _():
        fw_s[...] = jnp.dot(feat_ref[...], w_ref[...],
                            preferred_element_type=jnp.float32)

    t = jnp.dot(adj_ref[...], fw_s[...], preferred_element_type=jnp.float32)
    conv1 = jnp.maximum(t, 0.0)
    c1w2_ref[...] = jnp.dot(conv1, w2_ref[...],
                            preferred_element_type=jnp.float32)
    p1_ref[...] = jnp.dot(conv1, w1b_ref[...],
                          preferred_element_type=jnp.float32)


def _pass2_body(adj_ref, feat_ref, c1w2_ref, p1_ref,
                w_m1_ref, b_m1_ref, w_m2_ref, b_m2_ref,
                w1a_ref, w1c_ref, b1_ref, w2h_ref, b2_ref,
                out_ref):
    sm = jnp.maximum(jnp.dot(feat_ref[...], w_m1_ref[...],
                             preferred_element_type=jnp.float32)
                     + b_m1_ref[...], 0.0)
    self_c = jnp.dot(sm, w_m2_ref[...],
                     preferred_element_type=jnp.float32) + b_m2_ref[...]
    sacc = jnp.dot(self_c, w1a_ref[...], preferred_element_type=jnp.float32)

    conv2 = jnp.dot(adj_ref[...], c1w2_ref[...],
                    preferred_element_type=jnp.float32)
    z = jnp.maximum(
        sacc + p1_ref[...]
        + jnp.dot(conv2, w1c_ref[...], preferred_element_type=jnp.float32)
        + b1_ref[...], 0.0)
    out_ref[...] = jnp.dot(z, w2h_ref[...],
                           preferred_element_type=jnp.float32) + b2_ref[...]


@functools.partial(jax.jit, static_argnames=())
def kernel(features, adj, weight, weight2, w_m1, b_m1, w_m2, b_m2,
           w1, b1, w2, b2):
    n, d = features.shape
    h = weight.shape[1]
    o = weight2.shape[1]
    bi = 200
    grid = (n // bi,)

    w1a = w1[:h]
    w1b = w1[h:h + o]
    w1c = w1[h + o:]
    b_m1r = b_m1.reshape(1, -1)
    b_m2r = b_m2.reshape(1, -1)
    b1r = b1.reshape(1, -1)
    b2r = b2.reshape(1, -1)

    row_blk = pl.BlockSpec((bi, n), lambda i: (i, 0))
    feat_blk = pl.BlockSpec((bi, d), lambda i: (i, 0))
    out_blk = pl.BlockSpec((bi, h), lambda i: (i, 0))

    def full(a):
        return pl.BlockSpec(a.shape, lambda i: (0,) * a.ndim)

    c1w2, p1 = pl.pallas_call(
        _pass1_body,
        grid=grid,
        in_specs=[row_blk, full(features), full(weight), full(weight2),
                  full(w1b)],
        out_specs=[pl.BlockSpec((bi, o), lambda i: (i, 0)), out_blk],
        out_shape=[jax.ShapeDtypeStruct((n, o), jnp.float32),
                   jax.ShapeDtypeStruct((n, h), jnp.float32)],
        scratch_shapes=[pltpu.VMEM((n, h), jnp.float32)],
    )(adj, features, weight, weight2, w1b)

    out = pl.pallas_call(
        _pass2_body,
        grid=grid,
        in_specs=[row_blk, feat_blk, full(c1w2), out_blk,
                  full(w_m1), full(b_m1r), full(w_m2), full(b_m2r),
                  full(w1a), full(w1c), full(b1r), full(w2), full(b2r)],
        out_specs=pl.BlockSpec((bi, o), lambda i: (i, 0)),
        out_shape=jax.ShapeDtypeStruct((n, o), jnp.float32),
    )(adj, features, c1w2, p1, w_m1, b_m1r, w_m2, b_m2r,
      w1a, w1c, b1r, w2, b2r)
    return out
